# baseline (speedup 1.0000x reference)
import functools

import jax
import jax.numpy as jnp
from jax import lax
from jax.experimental import pallas as pl
from jax.experimental.pallas import tpu as pltpu
from jax.experimental.pallas import tpu_sc as plsc

NUM_ITEMS = 1000000
HIDDEN = 64
BATCH = 16384
NC, NS = 2, 16
NW = NC * NS
BPW = BATCH // NW
GRP = 16
NGRP = BPW // GRP
WIN = 8


def _gather_sc(idx, pt, ct):
    mesh = plsc.VectorSubcoreMesh(core_axis_name="c", subcore_axis_name="s")

    @functools.partial(
        pl.kernel,
        mesh=mesh,
        compiler_params=pltpu.CompilerParams(
            use_tc_tiling_on_sc=False, needs_layout_passes=False),
        out_type=(
            jax.ShapeDtypeStruct((HIDDEN, BATCH), jnp.float32),
            jax.ShapeDtypeStruct((HIDDEN, BATCH), jnp.float32),
        ),
        scratch_types=[
            pltpu.VMEM((BPW,), jnp.int32),
            pltpu.VMEM((HIDDEN, GRP * WIN), jnp.float32),
            pltpu.VMEM((HIDDEN, GRP * WIN), jnp.float32),
            pltpu.VMEM((HIDDEN, BPW), jnp.float32),
            pltpu.VMEM((HIDDEN, BPW), jnp.float32),
            pltpu.SemaphoreType.DMA,
            pltpu.SemaphoreType.DMA,
        ],
    )
    def k(idx_hbm, p_hbm, c_hbm, p_out, c_out, idx_vm,
          p_ring, c_ring, p_buf, c_buf, sem_p, sem_c):
        wid = lax.axis_index("c") * NS + lax.axis_index("s")
        base = wid * BPW
        pltpu.sync_copy(idx_hbm.at[pl.ds(base, BPW)], idx_vm)

        def group(g, carry):
            gvec = idx_vm[pl.ds(g * GRP, GRP)]
            waits = []
            for j in range(GRP):
                i8 = pl.multiple_of(jax.lax.shift_left(
                    jax.lax.shift_right_logical(gvec[j], 3), 3), 8)
                waits.append(pltpu.async_copy(
                    p_hbm.at[:, pl.ds(i8, WIN)],
                    p_ring.at[:, pl.ds(j * WIN, WIN)], sem_p))
                waits.append(pltpu.async_copy(
                    c_hbm.at[:, pl.ds(i8, WIN)],
                    c_ring.at[:, pl.ds(j * WIN, WIN)], sem_c))
            for w in waits:
                w.wait()
            for j in range(GRP):
                col = j * WIN + (gvec[j] & 7)
                cols = jnp.broadcast_to(col, (16,))
                e = g * GRP + j
                es = jnp.broadcast_to(e, (16,))
                for q in range(HIDDEN // 16):
                    rows = lax.iota(jnp.int32, 16) + 16 * q
                    plsc.store_scatter(
                        p_buf, [rows, es],
                        plsc.load_gather(p_ring, [rows, cols]))
                    plsc.store_scatter(
                        c_buf, [rows, es],
                        plsc.load_gather(c_ring, [rows, cols]))
            return carry

        lax.fori_loop(0, NGRP, group, 0)
        pltpu.sync_copy(p_buf, p_out.at[:, pl.ds(base, BPW)])
        pltpu.sync_copy(c_buf, c_out.at[:, pl.ds(base, BPW)])

    return k(idx, pt, ct)


def _rating_tc(pt, ct, W, b):
    blk = 4096

    def body(p_ref, c_ref, w_ref, b_ref, o_ref):
        s = jnp.sum((p_ref[...] + c_ref[...]) * w_ref[...], axis=0, keepdims=True)
        o_ref[...] = jax.nn.sigmoid(s + b_ref[...])

    return pl.pallas_call(
        body,
        grid=(BATCH // blk,),
        in_specs=[
            pl.BlockSpec((HIDDEN, blk), lambda i: (0, i)),
            pl.BlockSpec((HIDDEN, blk), lambda i: (0, i)),
            pl.BlockSpec((HIDDEN, 1), lambda i: (0, 0)),
            pl.BlockSpec((1, 1), lambda i: (0, 0)),
        ],
        out_specs=pl.BlockSpec((1, blk), lambda i: (0, i)),
        out_shape=jax.ShapeDtypeStruct((1, BATCH), jnp.float32),
    )(pt, ct, W.reshape(HIDDEN, 1), b.reshape(1, 1))


def kernel(item_indices, item_personality_table, item_commonality_table, W, b):
    idx = item_indices.astype(jnp.int32)
    pt_all, ct_all = _gather_sc(
        idx, item_personality_table.T, item_commonality_table.T)
    rating = _rating_tc(pt_all, ct_all, W, b).reshape(BATCH, 1)
    return (rating, pt_all.T, ct_all.T)
